# Initial kernel scaffold; baseline (speedup 1.0000x reference)
import jax
import jax.numpy as jnp
from jax.experimental import pallas as pl


def kernel(x, idxn, segment_ids, edgefeats, W1, b1, W2, b2):
    raise NotImplementedError("write your pallas kernel here")


# speedup vs baseline: 4.1945x; 4.1945x over previous
import functools

import jax
import jax.numpy as jnp
from jax import lax
from jax.experimental import pallas as pl
from jax.experimental.pallas import tpu as pltpu
from jax.experimental.pallas import tpu_sc as plsc

N_NODES = 10000
CIN = 16
COUT = 16
LANES = 128
G = 8
NCORES = 2
NSUB = 16
NW = NCORES * NSUB
BE = 2000
CH = 640
CH_LAST = N_NODES - (NSUB - 1) * CH
N_JUNK = N_NODES + 8


def _make_sc_kernels(rows_pad):
    gpw = rows_pad // (G * NW)
    assert gpw * G * NW == rows_pad
    mesh = plsc.VectorSubcoreMesh(core_axis_name="c", subcore_axis_name="s")
    params = pltpu.CompilerParams(use_tc_tiling_on_sc=False)

    @functools.partial(
        pl.kernel,
        out_type=jax.ShapeDtypeStruct((rows_pad, LANES, CIN), jnp.float32),
        mesh=mesh,
        scratch_types=[
            pltpu.VMEM((G, LANES), jnp.int32),
            pltpu.VMEM((G, LANES, CIN), jnp.float32),
            pltpu.SemaphoreType.DMA,
        ],
        compiler_params=params,
    )
    def gather_k(x_hbm, idx_hbm, out_hbm, idx_v, rows_v, sem):
        c = lax.axis_index("c")
        s = lax.axis_index("s")
        wid = c * NSUB + s

        @pl.loop(0, gpw)
        def _(g):
            r0 = (wid * gpw + g) * G
            pltpu.sync_copy(idx_hbm.at[pl.ds(r0, G)], idx_v)
            descs = [
                pltpu.async_copy(x_hbm.at[idx_v.at[j]], rows_v.at[j], sem)
                for j in range(G)
            ]
            for d in descs:
                d.wait()
            pltpu.sync_copy(rows_v, out_hbm.at[pl.ds(r0, G)])

    @functools.partial(
        pl.kernel,
        out_type=[
            jax.ShapeDtypeStruct((NCORES * N_NODES, CIN), jnp.float32),
            jax.ShapeDtypeStruct((NCORES * N_NODES, CIN), jnp.float32),
        ],
        mesh=mesh,
        scratch_types=[
            pltpu.VMEM((G, LANES), jnp.int32),
            pltpu.VMEM((G, LANES, CIN), jnp.float32),
            pltpu.VMEM((LANES, CIN), jnp.float32),
            pltpu.VMEM((CH, CIN), jnp.float32),
            pltpu.VMEM_SHARED((N_JUNK, CIN), jnp.float32),
            pltpu.VMEM_SHARED((N_JUNK, CIN), jnp.float32),
            pltpu.SemaphoreType.DMA,
        ],
        compiler_params=params,
    )
    def seg_k(seg_hbm, prod_hbm, sums_hbm, cnts_hbm,
              seg_v, pr_v, ones_v, zeros_v, acc_sh, cnt_sh, sem):
        c = lax.axis_index("c")
        s = lax.axis_index("s")
        wid = c * NSUB + s

        @pl.loop(0, CH)
        def _(i):
            zeros_v[i, :] = jnp.zeros((CIN,), jnp.float32)

        @pl.loop(0, LANES)
        def _(i):
            ones_v[i, :] = jnp.ones((CIN,), jnp.float32)

        rowbase = s * CH

        @pl.when(s < NSUB - 1)
        def _():
            pltpu.sync_copy(zeros_v, acc_sh.at[pl.ds(rowbase, CH)])
            pltpu.sync_copy(zeros_v, cnt_sh.at[pl.ds(rowbase, CH)])

        @pl.when(s == NSUB - 1)
        def _():
            pltpu.sync_copy(zeros_v.at[pl.ds(0, CH_LAST)],
                            acc_sh.at[pl.ds(rowbase, CH_LAST)])
            pltpu.sync_copy(zeros_v.at[pl.ds(0, CH_LAST)],
                            cnt_sh.at[pl.ds(rowbase, CH_LAST)])

        plsc.subcore_barrier()

        @pl.loop(0, gpw)
        def _(g):
            r0 = (wid * gpw + g) * G
            pltpu.sync_copy(seg_hbm.at[pl.ds(r0, G)], seg_v)
            pltpu.sync_copy(prod_hbm.at[pl.ds(r0, G)], pr_v)
            for j in range(G):
                pltpu.sync_copy(pr_v.at[j], acc_sh.at[seg_v.at[j]], add=True)
                pltpu.sync_copy(ones_v, cnt_sh.at[seg_v.at[j]], add=True)

        plsc.subcore_barrier()
        outbase = c * N_NODES + rowbase

        @pl.when(s < NSUB - 1)
        def _():
            pltpu.sync_copy(acc_sh.at[pl.ds(rowbase, CH)],
                            sums_hbm.at[pl.ds(outbase, CH)])
            pltpu.sync_copy(cnt_sh.at[pl.ds(rowbase, CH)],
                            cnts_hbm.at[pl.ds(outbase, CH)])

        @pl.when(s == NSUB - 1)
        def _():
            pltpu.sync_copy(acc_sh.at[pl.ds(rowbase, CH_LAST)],
                            sums_hbm.at[pl.ds(outbase, CH_LAST)])
            pltpu.sync_copy(cnt_sh.at[pl.ds(rowbase, CH_LAST)],
                            cnts_hbm.at[pl.ds(outbase, CH_LAST)])

    return gather_k, seg_k


def _dense_body(ef_ref, sel_ref, w1_ref, b1_ref, w2_ref, b2_ref,
                s_ref, r_ref, out_ref):
    h = jnp.maximum(
        jnp.dot(ef_ref[...], w1_ref[...], preferred_element_type=jnp.float32)
        + b1_ref[...], 0.0)
    w = jnp.dot(h, w2_ref[...], preferred_element_type=jnp.float32) + b2_ref[...]
    selrep = jnp.dot(sel_ref[...], s_ref[...], preferred_element_type=jnp.float32)
    out_ref[...] = jnp.dot(selrep * w, r_ref[...],
                           preferred_element_type=jnp.float32)


def _dense(ef, sel, W1, b1, W2, b2, S, R, E, E_pad):
    hid = W1.shape[1]
    grid = E // BE
    full = lambda i: (0, 0)
    return pl.pallas_call(
        _dense_body,
        grid=(grid,),
        in_specs=[
            pl.BlockSpec((BE, CIN), lambda i: (i, 0)),
            pl.BlockSpec((BE, CIN), lambda i: (i, 0)),
            pl.BlockSpec((CIN, hid), full),
            pl.BlockSpec((1, hid), full),
            pl.BlockSpec((hid, CIN * COUT), full),
            pl.BlockSpec((1, CIN * COUT), full),
            pl.BlockSpec((CIN, CIN * COUT), full),
            pl.BlockSpec((CIN * COUT, COUT), full),
        ],
        out_specs=pl.BlockSpec((BE, COUT), lambda i: (i, 0)),
        out_shape=jax.ShapeDtypeStruct((E_pad, COUT), jnp.float32),
    )(ef, sel, W1, b1, W2, b2, S, R)


def _combine_body(s_ref, c_ref, o_ref):
    sm = s_ref[0] + s_ref[1]
    ct = c_ref[0] + c_ref[1]
    o_ref[...] = jnp.where(ct > 0.0, sm / jnp.maximum(ct, 1.0), 0.0)


def _combine(sums2, cnts2):
    n2, nl = sums2.shape[1], sums2.shape[2]
    return pl.pallas_call(
        _combine_body,
        out_shape=jax.ShapeDtypeStruct((n2, nl), jnp.float32),
    )(sums2, cnts2)


def kernel(x, idxn, segment_ids, edgefeats, W1, b1, W2, b2):
    E = idxn.shape[0]
    chunk = G * LANES * NW
    E_pad = ((E + chunk - 1) // chunk) * chunk
    rows_pad = E_pad // LANES
    gather_k, seg_k = _make_sc_kernels(rows_pad)

    idx_pad = jnp.concatenate(
        [idxn, jnp.zeros((E_pad - E,), jnp.int32)]).reshape(rows_pad, LANES)
    seg_pad = jnp.concatenate(
        [segment_ids,
         jnp.full((E_pad - E,), N_NODES, jnp.int32)]).reshape(rows_pad, LANES)

    sel = gather_k(x, idx_pad).reshape(E_pad, CIN)

    S = (jnp.arange(CIN * COUT)[None, :] // COUT
         == jnp.arange(CIN)[:, None]).astype(jnp.float32)
    R = (jnp.arange(CIN * COUT)[:, None] % COUT
         == jnp.arange(COUT)[None, :]).astype(jnp.float32)
    products = _dense(edgefeats, sel, W1, b1.reshape(1, -1),
                      W2, b2.reshape(1, -1), S, R, E, E_pad)

    sums, cnts = seg_k(seg_pad, products.reshape(rows_pad, LANES, COUT))

    nl = 128
    n2 = N_NODES * COUT // nl
    out = _combine(sums.reshape(NCORES, n2, nl), cnts.reshape(NCORES, n2, nl))
    return out.reshape(N_NODES, COUT)


# speedup vs baseline: 4.8678x; 1.1605x over previous
import functools

import jax
import jax.numpy as jnp
from jax import lax
from jax.experimental import pallas as pl
from jax.experimental.pallas import tpu as pltpu
from jax.experimental.pallas import tpu_sc as plsc

N_NODES = 10000
CIN = 16
COUT = 16
LANES = 128
G = 8
NCORES = 2
NSUB = 16
NW = NCORES * NSUB
BL = 2560
CH = 640
CH_LAST = N_NODES - (NSUB - 1) * CH
N_JUNK = N_NODES + 8


def _make_sc_kernels(rows_pad):
    gpw = rows_pad // (G * NW)
    assert gpw * G * NW == rows_pad
    mesh = plsc.VectorSubcoreMesh(core_axis_name="c", subcore_axis_name="s")
    params = pltpu.CompilerParams(use_tc_tiling_on_sc=False)

    @functools.partial(
        pl.kernel,
        out_type=jax.ShapeDtypeStruct((rows_pad, LANES, CIN), jnp.float32),
        mesh=mesh,
        scratch_types=[
            pltpu.VMEM((G, LANES), jnp.int32),
            pltpu.VMEM((G, LANES, CIN), jnp.float32),
            pltpu.SemaphoreType.DMA,
        ],
        compiler_params=params,
    )
    def gather_k(x_hbm, idx_hbm, out_hbm, idx_v, rows_v, sem):
        c = lax.axis_index("c")
        s = lax.axis_index("s")
        wid = c * NSUB + s

        @pl.loop(0, gpw)
        def _(g):
            r0 = (wid * gpw + g) * G
            pltpu.sync_copy(idx_hbm.at[pl.ds(r0, G)], idx_v)
            descs = [
                pltpu.async_copy(x_hbm.at[idx_v.at[j]], rows_v.at[j], sem)
                for j in range(G)
            ]
            for d in descs:
                d.wait()
            pltpu.sync_copy(rows_v, out_hbm.at[pl.ds(r0, G)])

    @functools.partial(
        pl.kernel,
        out_type=[
            jax.ShapeDtypeStruct((NCORES * N_NODES, CIN), jnp.float32),
            jax.ShapeDtypeStruct((NCORES * N_NODES, CIN), jnp.float32),
        ],
        mesh=mesh,
        scratch_types=[
            pltpu.VMEM((G, LANES), jnp.int32),
            pltpu.VMEM((G, LANES, CIN), jnp.float32),
            pltpu.VMEM((LANES, CIN), jnp.float32),
            pltpu.VMEM((CH, CIN), jnp.float32),
            pltpu.VMEM_SHARED((N_JUNK, CIN), jnp.float32),
            pltpu.VMEM_SHARED((N_JUNK, CIN), jnp.float32),
            pltpu.SemaphoreType.DMA,
        ],
        compiler_params=params,
    )
    def seg_k(seg_hbm, prod_hbm, sums_hbm, cnts_hbm,
              seg_v, pr_v, ones_v, zeros_v, acc_sh, cnt_sh, sem):
        c = lax.axis_index("c")
        s = lax.axis_index("s")
        wid = c * NSUB + s

        @pl.loop(0, CH)
        def _(i):
            zeros_v[i, :] = jnp.zeros((CIN,), jnp.float32)

        @pl.loop(0, LANES)
        def _(i):
            ones_v[i, :] = jnp.ones((CIN,), jnp.float32)

        rowbase = s * CH

        @pl.when(s < NSUB - 1)
        def _():
            pltpu.sync_copy(zeros_v, acc_sh.at[pl.ds(rowbase, CH)])
            pltpu.sync_copy(zeros_v, cnt_sh.at[pl.ds(rowbase, CH)])

        @pl.when(s == NSUB - 1)
        def _():
            pltpu.sync_copy(zeros_v.at[pl.ds(0, CH_LAST)],
                            acc_sh.at[pl.ds(rowbase, CH_LAST)])
            pltpu.sync_copy(zeros_v.at[pl.ds(0, CH_LAST)],
                            cnt_sh.at[pl.ds(rowbase, CH_LAST)])

        plsc.subcore_barrier()

        @pl.loop(0, gpw)
        def _(g):
            r0 = (wid * gpw + g) * G
            pltpu.sync_copy(seg_hbm.at[pl.ds(r0, G)], seg_v)
            pltpu.sync_copy(prod_hbm.at[pl.ds(r0, G)], pr_v)
            for j in range(G):
                pltpu.sync_copy(pr_v.at[j], acc_sh.at[seg_v.at[j]], add=True)
                pltpu.sync_copy(ones_v, cnt_sh.at[seg_v.at[j]], add=True)

        plsc.subcore_barrier()
        outbase = c * N_NODES + rowbase

        @pl.when(s < NSUB - 1)
        def _():
            pltpu.sync_copy(acc_sh.at[pl.ds(rowbase, CH)],
                            sums_hbm.at[pl.ds(outbase, CH)])
            pltpu.sync_copy(cnt_sh.at[pl.ds(rowbase, CH)],
                            cnts_hbm.at[pl.ds(outbase, CH)])

        @pl.when(s == NSUB - 1)
        def _():
            pltpu.sync_copy(acc_sh.at[pl.ds(rowbase, CH_LAST)],
                            sums_hbm.at[pl.ds(outbase, CH_LAST)])
            pltpu.sync_copy(cnt_sh.at[pl.ds(rowbase, CH_LAST)],
                            cnts_hbm.at[pl.ds(outbase, CH_LAST)])

    return gather_k, seg_k


def _dense_body(eft_ref, selt_ref, w1t_ref, b1_ref, w2t_ref, b2_ref,
                st_ref, rt_ref, out_ref):
    ht = jnp.maximum(
        jnp.dot(w1t_ref[...], eft_ref[...], preferred_element_type=jnp.float32)
        + b1_ref[...], 0.0)
    wt = jnp.dot(w2t_ref[...], ht, preferred_element_type=jnp.float32) + b2_ref[...]
    selrept = jnp.dot(st_ref[...], selt_ref[...],
                      preferred_element_type=jnp.float32)
    out_ref[...] = jnp.dot(rt_ref[...], selrept * wt,
                           preferred_element_type=jnp.float32)


def _dense_t(eft, selt, W1t, b1, W2t, b2, St, Rt, E, E_pad):
    hid = W1t.shape[0]
    grid = E_pad // BL
    last = E // BL - 1
    full = lambda i: (0, 0)
    return pl.pallas_call(
        _dense_body,
        grid=(grid,),
        in_specs=[
            pl.BlockSpec((CIN, BL), lambda i: (0, jnp.minimum(i, last))),
            pl.BlockSpec((CIN, BL), lambda i: (0, i)),
            pl.BlockSpec((hid, CIN), full),
            pl.BlockSpec((hid, 1), full),
            pl.BlockSpec((CIN * COUT, hid), full),
            pl.BlockSpec((CIN * COUT, 1), full),
            pl.BlockSpec((CIN * COUT, CIN), full),
            pl.BlockSpec((COUT, CIN * COUT), full),
        ],
        out_specs=pl.BlockSpec((COUT, BL), lambda i: (0, i)),
        out_shape=jax.ShapeDtypeStruct((COUT, E_pad), jnp.float32),
    )(eft, selt, W1t, b1, W2t, b2, St, Rt)


def _combine_body(s_ref, c_ref, o_ref):
    sm = s_ref[0] + s_ref[1]
    ct = c_ref[0] + c_ref[1]
    o_ref[...] = jnp.where(ct > 0.0, sm / jnp.maximum(ct, 1.0), 0.0)


def _combine(sums2, cnts2):
    n2, nl = sums2.shape[1], sums2.shape[2]
    return pl.pallas_call(
        _combine_body,
        out_shape=jax.ShapeDtypeStruct((n2, nl), jnp.float32),
    )(sums2, cnts2)


def kernel(x, idxn, segment_ids, edgefeats, W1, b1, W2, b2):
    E = idxn.shape[0]
    chunk = G * LANES * NW
    E_pad = ((E + chunk - 1) // chunk) * chunk
    rows_pad = E_pad // LANES
    gather_k, seg_k = _make_sc_kernels(rows_pad)

    idx_pad = jnp.concatenate(
        [idxn, jnp.zeros((E_pad - E,), jnp.int32)]).reshape(rows_pad, LANES)
    seg_pad = jnp.concatenate(
        [segment_ids,
         jnp.full((E_pad - E,), N_NODES, jnp.int32)]).reshape(rows_pad, LANES)

    selt = gather_k(x, idx_pad).reshape(E_pad, CIN).T

    St = (jnp.arange(CIN * COUT)[:, None] // COUT
          == jnp.arange(CIN)[None, :]).astype(jnp.float32)
    Rt = (jnp.arange(CIN * COUT)[None, :] % COUT
          == jnp.arange(COUT)[:, None]).astype(jnp.float32)
    prod_t = _dense_t(edgefeats.T, selt, W1.T, b1.reshape(-1, 1),
                      W2.T, b2.reshape(-1, 1), St, Rt, E, E_pad)

    sums, cnts = seg_k(seg_pad, prod_t.T.reshape(rows_pad, LANES, COUT))

    nl = 128
    n2 = N_NODES * COUT // nl
    out = _combine(sums.reshape(NCORES, n2, nl), cnts.reshape(NCORES, n2, nl))
    return out.reshape(N_NODES, COUT)


# speedup vs baseline: 6.2198x; 1.2777x over previous
import functools

import jax
import jax.numpy as jnp
from jax import lax
from jax.experimental import pallas as pl
from jax.experimental.pallas import tpu as pltpu
from jax.experimental.pallas import tpu_sc as plsc

N_NODES = 10000
CIN = 16
COUT = 16
LANES = 128
G = 8
GL = G * LANES
NCORES = 2
NSUB = 16
NW = NCORES * NSUB
BL = 2560
CH = 640
CH_LAST = N_NODES - (NSUB - 1) * CH
N_JUNK = N_NODES + 8


def _make_sc_kernels(rows_pad):
    gpw = rows_pad // (G * NW)
    assert gpw * G * NW == rows_pad
    e_pad = rows_pad * LANES
    mesh = plsc.VectorSubcoreMesh(core_axis_name="c", subcore_axis_name="s")
    params = pltpu.CompilerParams(use_tc_tiling_on_sc=False,
                                  needs_layout_passes=False)

    @functools.partial(
        pl.kernel,
        out_type=jax.ShapeDtypeStruct((CIN, e_pad), jnp.float32),
        mesh=mesh,
        scratch_types=[
            pltpu.VMEM((G, LANES), jnp.int32),
            pltpu.VMEM((GL, CIN), jnp.float32),
            pltpu.VMEM((CIN, GL), jnp.float32),
            pltpu.SemaphoreType.DMA,
            pltpu.SemaphoreType.DMA,
        ],
        compiler_params=params,
    )
    def gather_k(x_hbm, idx_hbm, out_hbm, idx_v, rows_v, selt_v, sem, sem2):
        c = lax.axis_index("c")
        s = lax.axis_index("s")
        wid = c * NSUB + s
        lane_iota = lax.iota(jnp.int32, 16)

        @pl.loop(0, gpw)
        def _(g):
            r0 = (wid * gpw + g) * G
            e0 = r0 * LANES
            pltpu.sync_copy(idx_hbm.at[pl.ds(r0, G)], idx_v)
            descs = [
                pltpu.async_copy(x_hbm.at[idx_v.at[j]],
                                 rows_v.at[pl.ds(j * LANES, LANES)], sem)
                for j in range(G)
            ]
            for d in descs:
                d.wait()

            @pl.loop(0, GL // 16)
            def _(b):
                base = b * 16
                row_idx = base + lane_iota
                for i in range(16):
                    col_idx = jnp.full((16,), i, jnp.int32)
                    vec = plsc.load_gather(rows_v, [row_idx, col_idx])
                    selt_v[i, pl.ds(base, 16)] = vec

            descs2 = [
                pltpu.async_copy(selt_v.at[i], out_hbm.at[i, pl.ds(e0, GL)],
                                 sem2)
                for i in range(16)
            ]
            for d in descs2:
                d.wait()

    gather_k.__name__ = "sc_gather_transpose"

    @functools.partial(
        pl.kernel,
        out_type=[
            jax.ShapeDtypeStruct((NCORES * N_NODES, CIN), jnp.float32),
            jax.ShapeDtypeStruct((NCORES * N_NODES, CIN), jnp.float32),
        ],
        mesh=mesh,
        scratch_types=[
            pltpu.VMEM((G, LANES), jnp.int32),
            pltpu.VMEM((CIN, GL), jnp.float32),
            pltpu.VMEM((GL, CIN), jnp.float32),
            pltpu.VMEM((LANES, CIN), jnp.float32),
            pltpu.VMEM((CH, CIN), jnp.float32),
            pltpu.VMEM_SHARED((N_JUNK, CIN), jnp.float32),
            pltpu.VMEM_SHARED((N_JUNK, CIN), jnp.float32),
            pltpu.SemaphoreType.DMA,
        ],
        compiler_params=params,
    )
    def seg_k(seg_hbm, prodt_hbm, sums_hbm, cnts_hbm,
              seg_v, prt_v, pr_v, ones_v, zeros_v, acc_sh, cnt_sh, sem):
        c = lax.axis_index("c")
        s = lax.axis_index("s")
        wid = c * NSUB + s
        lane_iota = lax.iota(jnp.int32, 16)

        @pl.loop(0, CH)
        def _(i):
            zeros_v[i, :] = jnp.zeros((CIN,), jnp.float32)

        @pl.loop(0, LANES)
        def _(i):
            ones_v[i, :] = jnp.ones((CIN,), jnp.float32)

        rowbase = s * CH

        @pl.when(s < NSUB - 1)
        def _():
            pltpu.sync_copy(zeros_v, acc_sh.at[pl.ds(rowbase, CH)])
            pltpu.sync_copy(zeros_v, cnt_sh.at[pl.ds(rowbase, CH)])

        @pl.when(s == NSUB - 1)
        def _():
            pltpu.sync_copy(zeros_v.at[pl.ds(0, CH_LAST)],
                            acc_sh.at[pl.ds(rowbase, CH_LAST)])
            pltpu.sync_copy(zeros_v.at[pl.ds(0, CH_LAST)],
                            cnt_sh.at[pl.ds(rowbase, CH_LAST)])

        plsc.subcore_barrier()

        @pl.loop(0, gpw)
        def _(g):
            r0 = (wid * gpw + g) * G
            e0 = r0 * LANES
            pltpu.sync_copy(seg_hbm.at[pl.ds(r0, G)], seg_v)
            descs = [
                pltpu.async_copy(prodt_hbm.at[i, pl.ds(e0, GL)],
                                 prt_v.at[i], sem)
                for i in range(16)
            ]
            for d in descs:
                d.wait()

            @pl.loop(0, GL // 16)
            def _(b):
                base = b * 16
                row_idx = base + lane_iota
                for i in range(16):
                    col_idx = jnp.full((16,), i, jnp.int32)
                    vec = prt_v[i, pl.ds(base, 16)]
                    plsc.store_scatter(pr_v, [row_idx, col_idx], vec)

            for j in range(G):
                pltpu.sync_copy(pr_v.at[pl.ds(j * LANES, LANES)],
                                acc_sh.at[seg_v.at[j]], add=True)
                pltpu.sync_copy(ones_v, cnt_sh.at[seg_v.at[j]], add=True)

        plsc.subcore_barrier()
        outbase = c * N_NODES + rowbase

        @pl.when(s < NSUB - 1)
        def _():
            pltpu.sync_copy(acc_sh.at[pl.ds(rowbase, CH)],
                            sums_hbm.at[pl.ds(outbase, CH)])
            pltpu.sync_copy(cnt_sh.at[pl.ds(rowbase, CH)],
                            cnts_hbm.at[pl.ds(outbase, CH)])

        @pl.when(s == NSUB - 1)
        def _():
            pltpu.sync_copy(acc_sh.at[pl.ds(rowbase, CH_LAST)],
                            sums_hbm.at[pl.ds(outbase, CH_LAST)])
            pltpu.sync_copy(cnt_sh.at[pl.ds(rowbase, CH_LAST)],
                            cnts_hbm.at[pl.ds(outbase, CH_LAST)])

    seg_k.__name__ = "sc_segment_sum"
    return gather_k, seg_k


def _dense_body(eft_ref, selt_ref, w1t_ref, b1_ref, w2t_ref, b2_ref,
                st_ref, rt_ref, out_ref):
    ht = jnp.maximum(
        jnp.dot(w1t_ref[...], eft_ref[...], preferred_element_type=jnp.float32)
        + b1_ref[...], 0.0)
    wt = jnp.dot(w2t_ref[...], ht, preferred_element_type=jnp.float32) + b2_ref[...]
    selrept = jnp.dot(st_ref[...], selt_ref[...],
                      preferred_element_type=jnp.float32)
    out_ref[...] = jnp.dot(rt_ref[...], selrept * wt,
                           preferred_element_type=jnp.float32)


def _dense_t(eft, selt, W1t, b1, W2t, b2, St, Rt, E, E_pad):
    hid = W1t.shape[0]
    grid = E_pad // BL
    last = E // BL - 1
    full = lambda i: (0, 0)
    return pl.pallas_call(
        _dense_body,
        grid=(grid,),
        in_specs=[
            pl.BlockSpec((CIN, BL), lambda i: (0, jnp.minimum(i, last))),
            pl.BlockSpec((CIN, BL), lambda i: (0, i)),
            pl.BlockSpec((hid, CIN), full),
            pl.BlockSpec((hid, 1), full),
            pl.BlockSpec((CIN * COUT, hid), full),
            pl.BlockSpec((CIN * COUT, 1), full),
            pl.BlockSpec((CIN * COUT, CIN), full),
            pl.BlockSpec((COUT, CIN * COUT), full),
        ],
        out_specs=pl.BlockSpec((COUT, BL), lambda i: (0, i)),
        out_shape=jax.ShapeDtypeStruct((COUT, E_pad), jnp.float32),
    )(eft, selt, W1t, b1, W2t, b2, St, Rt)


def _combine_body(s_ref, c_ref, o_ref):
    sm = s_ref[0] + s_ref[1]
    ct = c_ref[0] + c_ref[1]
    o_ref[...] = jnp.where(ct > 0.0, sm / jnp.maximum(ct, 1.0), 0.0)


def _combine(sums2, cnts2):
    n2, nl = sums2.shape[1], sums2.shape[2]
    return pl.pallas_call(
        _combine_body,
        out_shape=jax.ShapeDtypeStruct((n2, nl), jnp.float32),
    )(sums2, cnts2)


def kernel(x, idxn, segment_ids, edgefeats, W1, b1, W2, b2):
    E = idxn.shape[0]
    chunk = GL * NW
    E_pad = ((E + chunk - 1) // chunk) * chunk
    rows_pad = E_pad // LANES
    gather_k, seg_k = _make_sc_kernels(rows_pad)

    idx_pad = jnp.concatenate(
        [idxn, jnp.zeros((E_pad - E,), jnp.int32)]).reshape(rows_pad, LANES)
    seg_pad = jnp.concatenate(
        [segment_ids,
         jnp.full((E_pad - E,), N_NODES, jnp.int32)]).reshape(rows_pad, LANES)

    selt = gather_k(x, idx_pad)

    St = (jnp.arange(CIN * COUT)[:, None] // COUT
          == jnp.arange(CIN)[None, :]).astype(jnp.float32)
    Rt = (jnp.arange(CIN * COUT)[None, :] % COUT
          == jnp.arange(COUT)[:, None]).astype(jnp.float32)
    prod_t = _dense_t(edgefeats.T, selt, W1.T, b1.reshape(-1, 1),
                      W2.T, b2.reshape(-1, 1), St, Rt, E, E_pad)

    sums, cnts = seg_k(seg_pad, prod_t)

    nl = 128
    n2 = N_NODES * COUT // nl
    out = _combine(sums.reshape(NCORES, n2, nl), cnts.reshape(NCORES, n2, nl))
    return out.reshape(N_NODES, COUT)


# speedup vs baseline: 6.7986x; 1.0931x over previous
import functools

import jax
import jax.numpy as jnp
from jax import lax
from jax.experimental import pallas as pl
from jax.experimental.pallas import tpu as pltpu
from jax.experimental.pallas import tpu_sc as plsc

N_NODES = 10000
CIN = 16
COUT = 16
LANES = 128
G = 8
GL = G * LANES
NCORES = 2
NSUB = 16
NW = NCORES * NSUB
BL = 2560
CH = 640
CH_LAST = N_NODES - (NSUB - 1) * CH
N_JUNK = N_NODES + 8


def _make_sc_kernels(rows_pad):
    gpw = rows_pad // (G * NW)
    assert gpw * G * NW == rows_pad
    e_pad = rows_pad * LANES
    mesh = plsc.VectorSubcoreMesh(core_axis_name="c", subcore_axis_name="s")
    params = pltpu.CompilerParams(use_tc_tiling_on_sc=False,
                                  needs_layout_passes=False)

    @functools.partial(
        pl.kernel,
        out_type=jax.ShapeDtypeStruct((CIN, e_pad), jnp.float32),
        mesh=mesh,
        scratch_types=[
            pltpu.VMEM((2, G, LANES), jnp.int32),
            pltpu.VMEM((2, GL, CIN), jnp.float32),
            pltpu.VMEM((2, CIN, GL), jnp.float32),
            pltpu.SemaphoreType.DMA,
            pltpu.SemaphoreType.DMA,
            pltpu.SemaphoreType.DMA,
        ],
        compiler_params=params,
    )
    def gather_k(x_hbm, idx_hbm, out_hbm, idx_v, rows_v, selt_v,
                 sem_i, sem_g, sem_o):
        c = lax.axis_index("c")
        s = lax.axis_index("s")
        wid = c * NSUB + s
        lane_iota = lax.iota(jnp.int32, 16)

        def fire_idx(g):
            r0 = (wid * gpw + g) * G
            return pltpu.async_copy(idx_hbm.at[pl.ds(r0, G)],
                                    idx_v.at[g & 1], sem_i)

        idx_descs = {0: fire_idx(0)}
        out_descs = {}
        for g in range(gpw):
            b = g & 1
            idx_descs.pop(g).wait()
            g_descs = [
                pltpu.async_copy(x_hbm.at[idx_v.at[b, j]],
                                 rows_v.at[b, pl.ds(j * LANES, LANES)],
                                 sem_g)
                for j in range(G)
            ]
            if g + 1 < gpw:
                idx_descs[g + 1] = fire_idx(g + 1)
            for d in g_descs:
                d.wait()
            if g >= 2:
                for d in out_descs.pop(g - 2):
                    d.wait()

            @pl.loop(0, GL // 16)
            def _(blk, b=b):
                base = blk * 16
                row_idx = base + lane_iota
                for i in range(16):
                    col_idx = jnp.full((16,), i, jnp.int32)
                    vec = plsc.load_gather(rows_v.at[b], [row_idx, col_idx])
                    selt_v[b, i, pl.ds(base, 16)] = vec

            e0 = (wid * gpw + g) * G * LANES
            out_descs[g] = [
                pltpu.async_copy(selt_v.at[b, i],
                                 out_hbm.at[i, pl.ds(e0, GL)], sem_o)
                for i in range(16)
            ]
        for g in sorted(out_descs):
            for d in out_descs[g]:
                d.wait()

    @functools.partial(
        pl.kernel,
        out_type=[
            jax.ShapeDtypeStruct((NCORES * N_NODES, CIN), jnp.float32),
            jax.ShapeDtypeStruct((NCORES * N_NODES, CIN), jnp.float32),
        ],
        mesh=mesh,
        scratch_types=[
            pltpu.VMEM((G, LANES), jnp.int32),
            pltpu.VMEM((CIN, GL), jnp.float32),
            pltpu.VMEM((GL, CIN), jnp.float32),
            pltpu.VMEM((LANES, CIN), jnp.float32),
            pltpu.VMEM((CH, CIN), jnp.float32),
            pltpu.VMEM_SHARED((N_JUNK, CIN), jnp.float32),
            pltpu.VMEM_SHARED((N_JUNK, CIN), jnp.float32),
            pltpu.SemaphoreType.DMA,
        ],
        compiler_params=params,
    )
    def seg_k(seg_hbm, prodt_hbm, sums_hbm, cnts_hbm,
              seg_v, prt_v, pr_v, ones_v, zeros_v, acc_sh, cnt_sh, sem):
        c = lax.axis_index("c")
        s = lax.axis_index("s")
        wid = c * NSUB + s
        lane_iota = lax.iota(jnp.int32, 16)

        @pl.loop(0, CH)
        def _(i):
            zeros_v[i, :] = jnp.zeros((CIN,), jnp.float32)

        @pl.loop(0, LANES)
        def _(i):
            ones_v[i, :] = jnp.ones((CIN,), jnp.float32)

        rowbase = s * CH

        @pl.when(s < NSUB - 1)
        def _():
            pltpu.sync_copy(zeros_v, acc_sh.at[pl.ds(rowbase, CH)])
            pltpu.sync_copy(zeros_v, cnt_sh.at[pl.ds(rowbase, CH)])

        @pl.when(s == NSUB - 1)
        def _():
            pltpu.sync_copy(zeros_v.at[pl.ds(0, CH_LAST)],
                            acc_sh.at[pl.ds(rowbase, CH_LAST)])
            pltpu.sync_copy(zeros_v.at[pl.ds(0, CH_LAST)],
                            cnt_sh.at[pl.ds(rowbase, CH_LAST)])

        plsc.subcore_barrier()

        @pl.loop(0, gpw)
        def _(g):
            r0 = (wid * gpw + g) * G
            e0 = r0 * LANES
            pltpu.sync_copy(seg_hbm.at[pl.ds(r0, G)], seg_v)
            descs = [
                pltpu.async_copy(prodt_hbm.at[i, pl.ds(e0, GL)],
                                 prt_v.at[i], sem)
                for i in range(16)
            ]
            for d in descs:
                d.wait()

            @pl.loop(0, GL // 16)
            def _(b):
                base = b * 16
                row_idx = base + lane_iota
                for i in range(16):
                    col_idx = jnp.full((16,), i, jnp.int32)
                    vec = prt_v[i, pl.ds(base, 16)]
                    plsc.store_scatter(pr_v, [row_idx, col_idx], vec)

            for j in range(G):
                pltpu.sync_copy(pr_v.at[pl.ds(j * LANES, LANES)],
                                acc_sh.at[seg_v.at[j]], add=True)
                pltpu.sync_copy(ones_v, cnt_sh.at[seg_v.at[j]], add=True)

        plsc.subcore_barrier()
        outbase = c * N_NODES + rowbase

        @pl.when(s < NSUB - 1)
        def _():
            pltpu.sync_copy(acc_sh.at[pl.ds(rowbase, CH)],
                            sums_hbm.at[pl.ds(outbase, CH)])
            pltpu.sync_copy(cnt_sh.at[pl.ds(rowbase, CH)],
                            cnts_hbm.at[pl.ds(outbase, CH)])

        @pl.when(s == NSUB - 1)
        def _():
            pltpu.sync_copy(acc_sh.at[pl.ds(rowbase, CH_LAST)],
                            sums_hbm.at[pl.ds(outbase, CH_LAST)])
            pltpu.sync_copy(cnt_sh.at[pl.ds(rowbase, CH_LAST)],
                            cnts_hbm.at[pl.ds(outbase, CH_LAST)])

    seg_k.__name__ = "sc_segment_sum"
    return gather_k, seg_k


def _dense_body(eft_ref, selt_ref, w1t_ref, b1_ref, w2t_ref, b2_ref, out_ref):
    ht = jnp.maximum(
        jnp.dot(w1t_ref[...], eft_ref[...], preferred_element_type=jnp.float32)
        + b1_ref[...], 0.0)
    wt = jnp.dot(w2t_ref[...], ht, preferred_element_type=jnp.float32) + b2_ref[...]
    selt = selt_ref[...]
    acc = selt[0:1, :] * wt[0:COUT, :]
    for i in range(1, CIN):
        acc += selt[i:i + 1, :] * wt[i * COUT:(i + 1) * COUT, :]
    out_ref[...] = acc


def _dense_t(eft, selt, W1t, b1, W2t, b2, E, E_pad):
    hid = W1t.shape[0]
    grid = E_pad // BL
    last = E // BL - 1
    full = lambda i: (0, 0)
    return pl.pallas_call(
        _dense_body,
        grid=(grid,),
        in_specs=[
            pl.BlockSpec((CIN, BL), lambda i: (0, jnp.minimum(i, last))),
            pl.BlockSpec((CIN, BL), lambda i: (0, i)),
            pl.BlockSpec((hid, CIN), full),
            pl.BlockSpec((hid, 1), full),
            pl.BlockSpec((CIN * COUT, hid), full),
            pl.BlockSpec((CIN * COUT, 1), full),
        ],
        out_specs=pl.BlockSpec((COUT, BL), lambda i: (0, i)),
        out_shape=jax.ShapeDtypeStruct((COUT, E_pad), jnp.float32),
    )(eft, selt, W1t, b1, W2t, b2)


def _combine_body(s_ref, c_ref, o_ref):
    sm = s_ref[0] + s_ref[1]
    ct = c_ref[0] + c_ref[1]
    o_ref[...] = jnp.where(ct > 0.0, sm / jnp.maximum(ct, 1.0), 0.0)


def _combine(sums2, cnts2):
    n2, nl = sums2.shape[1], sums2.shape[2]
    return pl.pallas_call(
        _combine_body,
        out_shape=jax.ShapeDtypeStruct((n2, nl), jnp.float32),
    )(sums2, cnts2)


def kernel(x, idxn, segment_ids, edgefeats, W1, b1, W2, b2):
    E = idxn.shape[0]
    chunk = GL * NW
    E_pad = ((E + chunk - 1) // chunk) * chunk
    rows_pad = E_pad // LANES
    gather_k, seg_k = _make_sc_kernels(rows_pad)

    idx_pad = jnp.concatenate(
        [idxn, jnp.zeros((E_pad - E,), jnp.int32)]).reshape(rows_pad, LANES)
    seg_pad = jnp.concatenate(
        [segment_ids,
         jnp.full((E_pad - E,), N_NODES, jnp.int32)]).reshape(rows_pad, LANES)

    selt = gather_k(x, idx_pad)

    prod_t = _dense_t(edgefeats.T, selt, W1.T, b1.reshape(-1, 1),
                      W2.T, b2.reshape(-1, 1), E, E_pad)

    sums, cnts = seg_k(seg_pad, prod_t)

    nl = 128
    n2 = N_NODES * COUT // nl
    out = _combine(sums.reshape(NCORES, n2, nl), cnts.reshape(NCORES, n2, nl))
    return out.reshape(N_NODES, COUT)


# speedup vs baseline: 7.1011x; 1.0445x over previous
import functools

import jax
import jax.numpy as jnp
from jax import lax
from jax.experimental import pallas as pl
from jax.experimental.pallas import tpu as pltpu
from jax.experimental.pallas import tpu_sc as plsc

N_NODES = 10000
CIN = 16
COUT = 16
LANES = 128
G = 8
GL = G * LANES
NCORES = 2
NSUB = 16
NW = NCORES * NSUB
BL = 2560
CH = 640
CH_LAST = N_NODES - (NSUB - 1) * CH
N_JUNK = N_NODES + 8


def _make_sc_kernels(rows_pad):
    gpw = rows_pad // (G * NW)
    assert gpw * G * NW == rows_pad
    e_pad = rows_pad * LANES
    mesh = plsc.VectorSubcoreMesh(core_axis_name="c", subcore_axis_name="s")
    params = pltpu.CompilerParams(use_tc_tiling_on_sc=False,
                                  needs_layout_passes=False)

    @functools.partial(
        pl.kernel,
        out_type=jax.ShapeDtypeStruct((CIN, e_pad), jnp.float32),
        mesh=mesh,
        scratch_types=[
            pltpu.VMEM((2, G, LANES), jnp.int32),
            pltpu.VMEM((2, GL, CIN), jnp.float32),
            pltpu.VMEM((2, CIN, GL), jnp.float32),
            pltpu.SemaphoreType.DMA,
            pltpu.SemaphoreType.DMA,
            pltpu.SemaphoreType.DMA,
        ],
        compiler_params=params,
    )
    def gather_k(x_hbm, idx_hbm, out_hbm, idx_v, rows_v, selt_v,
                 sem_i, sem_g, sem_o):
        c = lax.axis_index("c")
        s = lax.axis_index("s")
        wid = c * NSUB + s
        lane_iota = lax.iota(jnp.int32, 16)

        def fire_idx(g):
            r0 = (wid * gpw + g) * G
            return pltpu.async_copy(idx_hbm.at[pl.ds(r0, G)],
                                    idx_v.at[g & 1], sem_i)

        idx_descs = {0: fire_idx(0)}
        out_descs = {}
        for g in range(gpw):
            b = g & 1
            idx_descs.pop(g).wait()
            g_descs = [
                pltpu.async_copy(x_hbm.at[idx_v.at[b, j]],
                                 rows_v.at[b, pl.ds(j * LANES, LANES)],
                                 sem_g)
                for j in range(G)
            ]
            if g + 1 < gpw:
                idx_descs[g + 1] = fire_idx(g + 1)
            for d in g_descs:
                d.wait()
            if g >= 2:
                out_descs.pop(g - 2).wait()

            @pl.loop(0, GL // 16)
            def _(blk, b=b):
                base = blk * 16
                row_idx = base + lane_iota
                for i in range(16):
                    col_idx = jnp.full((16,), i, jnp.int32)
                    vec = plsc.load_gather(rows_v.at[b], [row_idx, col_idx])
                    selt_v[b, i, pl.ds(base, 16)] = vec

            e0 = (wid * gpw + g) * G * LANES
            out_descs[g] = pltpu.async_copy(
                selt_v.at[b], out_hbm.at[:, pl.ds(e0, GL)], sem_o)
        for g in sorted(out_descs):
            out_descs[g].wait()

    @functools.partial(
        pl.kernel,
        out_type=[
            jax.ShapeDtypeStruct((NCORES * N_NODES, CIN), jnp.float32),
            jax.ShapeDtypeStruct((NCORES * N_NODES, CIN), jnp.float32),
        ],
        mesh=mesh,
        scratch_types=[
            pltpu.VMEM((2, G, LANES), jnp.int32),
            pltpu.VMEM((2, CIN, GL), jnp.float32),
            pltpu.VMEM((2, GL, CIN), jnp.float32),
            pltpu.VMEM((LANES, CIN), jnp.float32),
            pltpu.VMEM((CH, CIN), jnp.float32),
            pltpu.VMEM_SHARED((N_JUNK, CIN), jnp.float32),
            pltpu.VMEM_SHARED((N_JUNK, CIN), jnp.float32),
            pltpu.SemaphoreType.DMA,
            pltpu.SemaphoreType.DMA,
        ],
        compiler_params=params,
    )
    def seg_k(seg_hbm, prodt_hbm, sums_hbm, cnts_hbm,
              seg_v, prt_v, pr_v, ones_v, zeros_v, acc_sh, cnt_sh,
              sem_i, sem_s):
        c = lax.axis_index("c")
        s = lax.axis_index("s")
        wid = c * NSUB + s
        lane_iota = lax.iota(jnp.int32, 16)

        @pl.loop(0, CH)
        def _(i):
            zeros_v[i, :] = jnp.zeros((CIN,), jnp.float32)

        @pl.loop(0, LANES)
        def _(i):
            ones_v[i, :] = jnp.ones((CIN,), jnp.float32)

        rowbase = s * CH

        @pl.when(s < NSUB - 1)
        def _():
            pltpu.sync_copy(zeros_v, acc_sh.at[pl.ds(rowbase, CH)])
            pltpu.sync_copy(zeros_v, cnt_sh.at[pl.ds(rowbase, CH)])

        @pl.when(s == NSUB - 1)
        def _():
            pltpu.sync_copy(zeros_v.at[pl.ds(0, CH_LAST)],
                            acc_sh.at[pl.ds(rowbase, CH_LAST)])
            pltpu.sync_copy(zeros_v.at[pl.ds(0, CH_LAST)],
                            cnt_sh.at[pl.ds(rowbase, CH_LAST)])

        plsc.subcore_barrier()

        def fire_stage(g):
            r0 = (wid * gpw + g) * G
            b = g & 1
            return [
                pltpu.async_copy(seg_hbm.at[pl.ds(r0, G)], seg_v.at[b],
                                 sem_i),
                pltpu.async_copy(prodt_hbm.at[:, pl.ds(r0 * LANES, GL)],
                                 prt_v.at[b], sem_i),
            ]

        stage_descs = {0: fire_stage(0)}
        scat_descs = {}
        for g in range(gpw):
            b = g & 1
            for d in stage_descs.pop(g):
                d.wait()
            if g >= 1:
                for d in scat_descs.pop(g - 1):
                    d.wait()
            if g + 1 < gpw:
                stage_descs[g + 1] = fire_stage(g + 1)

            @pl.loop(0, GL // 16)
            def _(blk, b=b):
                base = blk * 16
                row_idx = base + lane_iota
                for i in range(16):
                    col_idx = jnp.full((16,), i, jnp.int32)
                    vec = prt_v[b, i, pl.ds(base, 16)]
                    plsc.store_scatter(pr_v.at[b], [row_idx, col_idx], vec)

            descs = []
            for j in range(G):
                descs.append(pltpu.async_copy(
                    pr_v.at[b, pl.ds(j * LANES, LANES)],
                    acc_sh.at[seg_v.at[b, j]], sem_s, add=True))
                descs.append(pltpu.async_copy(
                    ones_v, cnt_sh.at[seg_v.at[b, j]], sem_s, add=True))
            scat_descs[g] = descs
        for g in sorted(scat_descs):
            for d in scat_descs[g]:
                d.wait()

        plsc.subcore_barrier()
        outbase = c * N_NODES + rowbase

        @pl.when(s < NSUB - 1)
        def _():
            pltpu.sync_copy(acc_sh.at[pl.ds(rowbase, CH)],
                            sums_hbm.at[pl.ds(outbase, CH)])
            pltpu.sync_copy(cnt_sh.at[pl.ds(rowbase, CH)],
                            cnts_hbm.at[pl.ds(outbase, CH)])

        @pl.when(s == NSUB - 1)
        def _():
            pltpu.sync_copy(acc_sh.at[pl.ds(rowbase, CH_LAST)],
                            sums_hbm.at[pl.ds(outbase, CH_LAST)])
            pltpu.sync_copy(cnt_sh.at[pl.ds(rowbase, CH_LAST)],
                            cnts_hbm.at[pl.ds(outbase, CH_LAST)])

    seg_k.__name__ = "sc_segment_sum"
    return gather_k, seg_k


def _dense_body(eft_ref, selt_ref, w1t_ref, b1_ref, w2t_ref, b2_ref, out_ref):
    ht = jnp.maximum(
        jnp.dot(w1t_ref[...], eft_ref[...], preferred_element_type=jnp.float32)
        + b1_ref[...], 0.0)
    wt = jnp.dot(w2t_ref[...], ht, preferred_element_type=jnp.float32) + b2_ref[...]
    selt = selt_ref[...]
    acc = selt[0:1, :] * wt[0:COUT, :]
    for i in range(1, CIN):
        acc += selt[i:i + 1, :] * wt[i * COUT:(i + 1) * COUT, :]
    out_ref[...] = acc


def _dense_t(eft, selt, W1t, b1, W2t, b2, E, E_pad):
    hid = W1t.shape[0]
    grid = E_pad // BL
    last = E // BL - 1
    full = lambda i: (0, 0)
    return pl.pallas_call(
        _dense_body,
        grid=(grid,),
        in_specs=[
            pl.BlockSpec((CIN, BL), lambda i: (0, jnp.minimum(i, last))),
            pl.BlockSpec((CIN, BL), lambda i: (0, i)),
            pl.BlockSpec((hid, CIN), full),
            pl.BlockSpec((hid, 1), full),
            pl.BlockSpec((CIN * COUT, hid), full),
            pl.BlockSpec((CIN * COUT, 1), full),
        ],
        out_specs=pl.BlockSpec((COUT, BL), lambda i: (0, i)),
        out_shape=jax.ShapeDtypeStruct((COUT, E_pad), jnp.float32),
    )(eft, selt, W1t, b1, W2t, b2)


def _combine_body(s_ref, c_ref, o_ref):
    sm = s_ref[0] + s_ref[1]
    ct = c_ref[0] + c_ref[1]
    o_ref[...] = jnp.where(ct > 0.0, sm / jnp.maximum(ct, 1.0), 0.0)


def _combine(sums2, cnts2):
    n2, nl = sums2.shape[1], sums2.shape[2]
    return pl.pallas_call(
        _combine_body,
        out_shape=jax.ShapeDtypeStruct((n2, nl), jnp.float32),
    )(sums2, cnts2)


def kernel(x, idxn, segment_ids, edgefeats, W1, b1, W2, b2):
    E = idxn.shape[0]
    chunk = GL * NW
    E_pad = ((E + chunk - 1) // chunk) * chunk
    rows_pad = E_pad // LANES
    gather_k, seg_k = _make_sc_kernels(rows_pad)

    idx_pad = jnp.concatenate(
        [idxn, jnp.zeros((E_pad - E,), jnp.int32)]).reshape(rows_pad, LANES)
    seg_pad = jnp.concatenate(
        [segment_ids,
         jnp.full((E_pad - E,), N_NODES, jnp.int32)]).reshape(rows_pad, LANES)

    selt = gather_k(x, idx_pad)

    prod_t = _dense_t(edgefeats.T, selt, W1.T, b1.reshape(-1, 1),
                      W2.T, b2.reshape(-1, 1), E, E_pad)

    sums, cnts = seg_k(seg_pad, prod_t)

    nl = 128
    n2 = N_NODES * COUT // nl
    out = _combine(sums.reshape(NCORES, n2, nl), cnts.reshape(NCORES, n2, nl))
    return out.reshape(N_NODES, COUT)


# speedup vs baseline: 8.1921x; 1.1536x over previous
import functools

import jax
import jax.numpy as jnp
from jax import lax
from jax.experimental import pallas as pl
from jax.experimental.pallas import tpu as pltpu
from jax.experimental.pallas import tpu_sc as plsc

N_NODES = 10000
CIN = 16
COUT = 16
LANES = 128
G = 8
GL = G * LANES
NCORES = 2
NSUB = 16
NW = NCORES * NSUB
BL = 2560
CH = 640
CH_LAST = N_NODES - (NSUB - 1) * CH
N_JUNK = N_NODES + 8


def _make_sc_kernels(rows_pad):
    gpw = rows_pad // (G * NW)
    assert gpw * G * NW == rows_pad
    e_pad = rows_pad * LANES
    mesh = plsc.VectorSubcoreMesh(core_axis_name="c", subcore_axis_name="s")
    params = pltpu.CompilerParams(use_tc_tiling_on_sc=False,
                                  needs_layout_passes=False)

    @functools.partial(
        pl.kernel,
        out_type=jax.ShapeDtypeStruct((CIN, e_pad), jnp.float32),
        mesh=mesh,
        scratch_types=[
            pltpu.VMEM((gpw * G, LANES), jnp.int32),
            pltpu.VMEM((2, GL, CIN), jnp.float32),
            pltpu.VMEM((2, CIN, GL), jnp.float32),
            pltpu.SemaphoreType.DMA,
            pltpu.SemaphoreType.DMA,
        ],
        compiler_params=params,
    )
    def gather_k(x_hbm, idx_hbm, out_hbm, idx_v, rows_v, selt_v,
                 sem_g, sem_o):
        c = lax.axis_index("c")
        s = lax.axis_index("s")
        wid = c * NSUB + s
        lane_iota = lax.iota(jnp.int32, 16)

        pltpu.sync_copy(idx_hbm.at[pl.ds(wid * gpw * G, gpw * G)], idx_v)

        def fire_gathers(g):
            b = g & 1
            return [
                pltpu.async_copy(x_hbm.at[idx_v.at[g * G + j]],
                                 rows_v.at[b, pl.ds(j * LANES, LANES)],
                                 sem_g)
                for j in range(G)
            ]

        gat_descs = {0: fire_gathers(0)}
        out_descs = {}
        for g in range(gpw):
            b = g & 1
            for d in gat_descs.pop(g):
                d.wait()
            if g + 1 < gpw:
                gat_descs[g + 1] = fire_gathers(g + 1)
            if g >= 2:
                out_descs.pop(g - 2).wait()

            @pl.loop(0, GL // 16)
            def _(blk, b=b):
                base = blk * 16
                row_idx = base + lane_iota
                for i in range(16):
                    col_idx = jnp.full((16,), i, jnp.int32)
                    vec = plsc.load_gather(rows_v.at[b], [row_idx, col_idx])
                    selt_v[b, i, pl.ds(base, 16)] = vec

            e0 = (wid * gpw + g) * G * LANES
            out_descs[g] = pltpu.async_copy(
                selt_v.at[b], out_hbm.at[:, pl.ds(e0, GL)], sem_o)
        for g in sorted(out_descs):
            out_descs[g].wait()

    @functools.partial(
        pl.kernel,
        out_type=[
            jax.ShapeDtypeStruct((NCORES * N_NODES, CIN), jnp.float32),
            jax.ShapeDtypeStruct((NCORES * N_NODES, CIN), jnp.float32),
        ],
        mesh=mesh,
        scratch_types=[
            pltpu.VMEM((gpw * G, LANES), jnp.int32),
            pltpu.VMEM((2, CIN, GL), jnp.float32),
            pltpu.VMEM((2, GL, CIN), jnp.float32),
            pltpu.VMEM((LANES, CIN), jnp.float32),
            pltpu.VMEM((CH, CIN), jnp.float32),
            pltpu.VMEM_SHARED((N_JUNK, CIN), jnp.float32),
            pltpu.VMEM_SHARED((N_JUNK, CIN), jnp.float32),
            pltpu.SemaphoreType.DMA,
            pltpu.SemaphoreType.DMA,
        ],
        compiler_params=params,
    )
    def seg_k(seg_hbm, prodt_hbm, sums_hbm, cnts_hbm,
              seg_v, prt_v, pr_v, ones_v, zeros_v, acc_sh, cnt_sh,
              sem_i, sem_s):
        c = lax.axis_index("c")
        s = lax.axis_index("s")
        wid = c * NSUB + s
        lane_iota = lax.iota(jnp.int32, 16)

        @pl.loop(0, CH)
        def _(i):
            zeros_v[i, :] = jnp.zeros((CIN,), jnp.float32)

        @pl.loop(0, LANES)
        def _(i):
            ones_v[i, :] = jnp.ones((CIN,), jnp.float32)

        rowbase = s * CH

        @pl.when(s < NSUB - 1)
        def _():
            pltpu.sync_copy(zeros_v, acc_sh.at[pl.ds(rowbase, CH)])
            pltpu.sync_copy(zeros_v, cnt_sh.at[pl.ds(rowbase, CH)])

        @pl.when(s == NSUB - 1)
        def _():
            pltpu.sync_copy(zeros_v.at[pl.ds(0, CH_LAST)],
                            acc_sh.at[pl.ds(rowbase, CH_LAST)])
            pltpu.sync_copy(zeros_v.at[pl.ds(0, CH_LAST)],
                            cnt_sh.at[pl.ds(rowbase, CH_LAST)])

        plsc.subcore_barrier()
        pltpu.sync_copy(seg_hbm.at[pl.ds(wid * gpw * G, gpw * G)], seg_v)

        def fire_stage(g):
            r0 = (wid * gpw + g) * G
            return pltpu.async_copy(
                prodt_hbm.at[:, pl.ds(r0 * LANES, GL)], prt_v.at[g & 1],
                sem_i)

        stage_descs = {0: fire_stage(0)}
        scat_descs = {}
        for g in range(gpw):
            b = g & 1
            stage_descs.pop(g).wait()
            if g >= 1:
                for d in scat_descs.pop(g - 1):
                    d.wait()
            if g + 1 < gpw:
                stage_descs[g + 1] = fire_stage(g + 1)

            @pl.loop(0, GL // 16)
            def _(blk, b=b):
                base = blk * 16
                row_idx = base + lane_iota
                for i in range(16):
                    col_idx = jnp.full((16,), i, jnp.int32)
                    vec = prt_v[b, i, pl.ds(base, 16)]
                    plsc.store_scatter(pr_v.at[b], [row_idx, col_idx], vec)

            descs = []
            for j in range(G):
                descs.append(pltpu.async_copy(
                    pr_v.at[b, pl.ds(j * LANES, LANES)],
                    acc_sh.at[seg_v.at[g * G + j]], sem_s, add=True))
                descs.append(pltpu.async_copy(
                    ones_v, cnt_sh.at[seg_v.at[g * G + j]], sem_s, add=True))
            scat_descs[g] = descs
        for g in sorted(scat_descs):
            for d in scat_descs[g]:
                d.wait()

        plsc.subcore_barrier()
        outbase = c * N_NODES + rowbase

        @pl.when(s < NSUB - 1)
        def _():
            pltpu.sync_copy(acc_sh.at[pl.ds(rowbase, CH)],
                            sums_hbm.at[pl.ds(outbase, CH)])
            pltpu.sync_copy(cnt_sh.at[pl.ds(rowbase, CH)],
                            cnts_hbm.at[pl.ds(outbase, CH)])

        @pl.when(s == NSUB - 1)
        def _():
            pltpu.sync_copy(acc_sh.at[pl.ds(rowbase, CH_LAST)],
                            sums_hbm.at[pl.ds(outbase, CH_LAST)])
            pltpu.sync_copy(cnt_sh.at[pl.ds(rowbase, CH_LAST)],
                            cnts_hbm.at[pl.ds(outbase, CH_LAST)])

    seg_k.__name__ = "sc_segment_sum"
    return gather_k, seg_k


def _dense_body(eft_ref, selt_ref, w1t_ref, b1_ref, w2t_ref, b2_ref, out_ref):
    ht = jnp.maximum(
        jnp.dot(w1t_ref[...], eft_ref[...], preferred_element_type=jnp.float32)
        + b1_ref[...], 0.0)
    wt = jnp.dot(w2t_ref[...], ht, preferred_element_type=jnp.float32) + b2_ref[...]
    selt = selt_ref[...]
    acc = selt[0:1, :] * wt[0:COUT, :]
    for i in range(1, CIN):
        acc += selt[i:i + 1, :] * wt[i * COUT:(i + 1) * COUT, :]
    out_ref[...] = acc


def _dense_t(eft, selt, W1t, b1, W2t, b2, E, E_pad):
    hid = W1t.shape[0]
    grid = E_pad // BL
    last = E // BL - 1
    full = lambda i: (0, 0)
    return pl.pallas_call(
        _dense_body,
        grid=(grid,),
        in_specs=[
            pl.BlockSpec((CIN, BL), lambda i: (0, jnp.minimum(i, last))),
            pl.BlockSpec((CIN, BL), lambda i: (0, i)),
            pl.BlockSpec((hid, CIN), full),
            pl.BlockSpec((hid, 1), full),
            pl.BlockSpec((CIN * COUT, hid), full),
            pl.BlockSpec((CIN * COUT, 1), full),
        ],
        out_specs=pl.BlockSpec((COUT, BL), lambda i: (0, i)),
        out_shape=jax.ShapeDtypeStruct((COUT, E_pad), jnp.float32),
    )(eft, selt, W1t, b1, W2t, b2)


def _combine_body(s_ref, c_ref, o_ref):
    sm = s_ref[0] + s_ref[1]
    ct = c_ref[0] + c_ref[1]
    o_ref[...] = jnp.where(ct > 0.0, sm / jnp.maximum(ct, 1.0), 0.0)


def _combine(sums2, cnts2):
    n2, nl = sums2.shape[1], sums2.shape[2]
    return pl.pallas_call(
        _combine_body,
        out_shape=jax.ShapeDtypeStruct((n2, nl), jnp.float32),
    )(sums2, cnts2)


def kernel(x, idxn, segment_ids, edgefeats, W1, b1, W2, b2):
    E = idxn.shape[0]
    chunk = GL * NW
    E_pad = ((E + chunk - 1) // chunk) * chunk
    rows_pad = E_pad // LANES
    gather_k, seg_k = _make_sc_kernels(rows_pad)

    idx_pad = jnp.concatenate(
        [idxn, jnp.zeros((E_pad - E,), jnp.int32)]).reshape(rows_pad, LANES)
    seg_pad = jnp.concatenate(
        [segment_ids,
         jnp.full((E_pad - E,), N_NODES, jnp.int32)]).reshape(rows_pad, LANES)

    selt = gather_k(x, idx_pad)

    prod_t = _dense_t(edgefeats.T, selt, W1.T, b1.reshape(-1, 1),
                      W2.T, b2.reshape(-1, 1), E, E_pad)

    sums, cnts = seg_k(seg_pad, prod_t)

    nl = 128
    n2 = N_NODES * COUT // nl
    out = _combine(sums.reshape(NCORES, n2, nl), cnts.reshape(NCORES, n2, nl))
    return out.reshape(N_NODES, COUT)
